# Initial kernel scaffold; baseline (speedup 1.0000x reference)
import jax
import jax.numpy as jnp
from jax.experimental import pallas as pl


def kernel(x, gate_w, cfc_w, cfc_b, cproj_w, cproj_b):
    raise NotImplementedError("write your pallas kernel here")


# speedup vs baseline: 3.2511x; 3.2511x over previous
import functools

import jax
import jax.numpy as jnp
from jax import lax
from jax.experimental import pallas as pl
from jax.experimental.pallas import tpu as pltpu
from jax.experimental.pallas import tpu_sc as plsc

N = 8192
C = 1024
E = 8
DFF = 4096
CAP = 1536

BN = 1024
NB = N // BN

BM = 512
M = CAP // BM
BK = 1024
K = DFF // BK
RD = BM + E * CAP
EB = RD // BM

NSC, NSUB = 2, 16
NW = NSC * NSUB
TPW = N // NW
CHUNK = 64


def _router_body(x_ref, gw_ref, g_ref, w_ref, stats_ref):
    i = pl.program_id(0)

    @pl.when(i == 0)
    def _():
        stats_ref[...] = jnp.zeros((8, 8), jnp.float32)

    xb = x_ref[...]
    logits = jnp.dot(xb, gw_ref[...], preferred_element_type=jnp.float32)

    l2 = logits / 1.66
    m2 = jnp.max(l2, axis=-1, keepdims=True)
    e2 = jnp.exp(l2 - m2)
    p2 = e2 / jnp.sum(e2, axis=-1, keepdims=True)
    s_p1mp = jnp.sum(p2 * (1.0 - p2))
    sum_p2 = jnp.sum(p2, axis=0, keepdims=True)

    lm = jnp.max(logits, axis=-1, keepdims=True)
    el = jnp.exp(logits - lm)
    se = jnp.sum(el, axis=-1, keepdims=True)
    lse = lm + jnp.log(se)
    s_lse2 = jnp.sum(lse * lse)

    rp = el / se
    w = jnp.max(rp, axis=-1)
    iota_e = lax.broadcasted_iota(jnp.int32, (BN, E), 1)
    eq = rp == w[:, None]
    idx = jnp.min(jnp.where(eq, iota_e, E), axis=-1)
    onehot = (iota_e == idx[:, None]).astype(jnp.float32)

    sum_rp = jnp.sum(rp, axis=0, keepdims=True)
    sum_w = jnp.sum(onehot * w[:, None], axis=0, keepdims=True)
    blk_hist = jnp.sum(onehot, axis=0, keepdims=True)

    r_io = lax.broadcasted_iota(jnp.int32, (BN, BN), 0)
    c_io = lax.broadcasted_iota(jnp.int32, (BN, BN), 1)
    tri = (r_io >= c_io).astype(jnp.float32)
    incl = jnp.dot(tri, onehot, preferred_element_type=jnp.float32)

    offs = stats_ref[3:4, :]
    rank = incl + offs
    rank_t = jnp.sum(rank * onehot, axis=-1)
    kept = rank_t <= CAP
    slot = jnp.where(kept, BM + idx * CAP + rank_t.astype(jnp.int32) - 1, 0)

    g_ref[...] = slot.reshape(1, 1, BN)
    w_ref[...] = w.reshape(1, 1, BN)

    stats_ref[0:1, :] += sum_p2
    stats_ref[1:2, :] += sum_rp
    stats_ref[2:3, :] += sum_w
    stats_ref[3:4, :] = offs + blk_hist
    stats_ref[4:5, :] += jnp.full((1, 8), s_p1mp, jnp.float32)
    stats_ref[5:6, :] += jnp.full((1, 8), s_lse2, jnp.float32)


def _router(x_flat, gate_w):
    return pl.pallas_call(
        _router_body,
        grid=(NB,),
        in_specs=[
            pl.BlockSpec((BN, C), lambda i: (i, 0)),
            pl.BlockSpec((C, E), lambda i: (0, 0)),
        ],
        out_specs=[
            pl.BlockSpec((1, 1, BN), lambda i: (i, 0, 0)),
            pl.BlockSpec((1, 1, BN), lambda i: (i, 0, 0)),
            pl.BlockSpec((8, 8), lambda i: (0, 0)),
        ],
        out_shape=[
            jax.ShapeDtypeStruct((NB, 1, BN), jnp.int32),
            jax.ShapeDtypeStruct((NB, 1, BN), jnp.float32),
            jax.ShapeDtypeStruct((8, 8), jnp.float32),
        ],
    )(x_flat, gate_w)


def _dispatch(x_flat, g, w):
    mesh = plsc.VectorSubcoreMesh(
        core_axis_name="c", subcore_axis_name="s",
        num_cores=NSC, num_subcores=NSUB)

    @functools.partial(
        pl.kernel, mesh=mesh,
        out_type=[
            jax.ShapeDtypeStruct((RD, C), jnp.float32),
            jax.ShapeDtypeStruct((RD,), jnp.float32),
        ],
        scratch_types=[
            pltpu.VMEM((CHUNK,), jnp.int32),
            pltpu.VMEM((CHUNK, C), jnp.float32),
            pltpu.VMEM((CHUNK,), jnp.float32),
            pltpu.SemaphoreType.DMA,
        ],
    )
    def run(x_hbm, g_hbm, w_hbm, d_out, ws_out, idx_v, rows_v, wv, sem):
        wid = lax.axis_index("s") * NSC + lax.axis_index("c")
        base = wid * TPW
        for cb in range(TPW // CHUNK):
            off = base + cb * CHUNK
            pltpu.sync_copy(g_hbm.at[pl.ds(off, CHUNK)], idx_v)
            pltpu.sync_copy(x_hbm.at[pl.ds(off, CHUNK)], rows_v)
            pltpu.async_copy(rows_v, d_out.at[idx_v], sem).wait()
            pltpu.sync_copy(w_hbm.at[pl.ds(off, CHUNK)], wv)
            pltpu.async_copy(wv, ws_out.at[idx_v], sem).wait()

    return run(x_flat, g, w)


def _ffn_body(counts_ref, d_ref, w1_ref, b1_ref, w2_ref, b2_ref, ws_ref, y_ref):
    b = pl.program_id(0)
    k = pl.program_id(1)
    e = jnp.clip((b - 1) // M, 0, E - 1)
    m = (b - 1) % M
    active = jnp.logical_and(b >= 1, m * BM < counts_ref[e])

    @pl.when(active)
    def _():
        x = d_ref[...]
        h = jnp.dot(x, w1_ref[0], preferred_element_type=jnp.float32)
        h = h + b1_ref[0]
        h = jax.nn.gelu(h, approximate=True)
        contrib = jnp.dot(h, w2_ref[0], preferred_element_type=jnp.float32)

        @pl.when(k == 0)
        def _():
            y_ref[...] = contrib

        @pl.when(k != 0)
        def _():
            y_ref[...] += contrib

        @pl.when(k == K - 1)
        def _():
            y_ref[...] = (y_ref[...] + b2_ref[0]) * ws_ref[...]

    @pl.when(jnp.logical_and(jnp.logical_not(active), k == K - 1))
    def _():
        y_ref[...] = jnp.zeros((BM, C), jnp.float32)


def _ffn(counts, d, cfc_w, cfc_b, cproj_w, cproj_b, ws2d):
    grid_spec = pltpu.PrefetchScalarGridSpec(
        num_scalar_prefetch=1,
        grid=(EB, K),
        in_specs=[
            pl.BlockSpec((BM, C), lambda b, k, cnt: (b, 0)),
            pl.BlockSpec((1, C, BK), lambda b, k, cnt: (lax.div(b - 1, M), 0, k)),
            pl.BlockSpec((1, 1, BK), lambda b, k, cnt: (lax.div(b - 1, M), 0, k)),
            pl.BlockSpec((1, BK, C), lambda b, k, cnt: (lax.div(b - 1, M), k, 0)),
            pl.BlockSpec((1, 1, C), lambda b, k, cnt: (lax.div(b - 1, M), 0, 0)),
            pl.BlockSpec((BM, 1), lambda b, k, cnt: (b, 0)),
        ],
        out_specs=pl.BlockSpec((BM, C), lambda b, k, cnt: (b, 0)),
    )
    return pl.pallas_call(
        _ffn_body,
        grid_spec=grid_spec,
        out_shape=jax.ShapeDtypeStruct((RD, C), jnp.float32),
        compiler_params=pltpu.CompilerParams(
            dimension_semantics=("arbitrary", "arbitrary")),
    )(counts, d, cfc_w, cfc_b.reshape(E, 1, DFF), cproj_w,
      cproj_b.reshape(E, 1, C), ws2d)


def _combine(y, g):
    mesh = plsc.VectorSubcoreMesh(
        core_axis_name="c", subcore_axis_name="s",
        num_cores=NSC, num_subcores=NSUB)

    @functools.partial(
        pl.kernel, mesh=mesh,
        out_type=jax.ShapeDtypeStruct((N, C), jnp.float32),
        scratch_types=[
            pltpu.VMEM((CHUNK,), jnp.int32),
            pltpu.VMEM((CHUNK, C), jnp.float32),
            pltpu.SemaphoreType.DMA,
        ],
    )
    def run(y_hbm, g_hbm, out_hbm, idx_v, rows_v, sem):
        wid = lax.axis_index("s") * NSC + lax.axis_index("c")
        base = wid * TPW
        for cb in range(TPW // CHUNK):
            off = base + cb * CHUNK
            pltpu.sync_copy(g_hbm.at[pl.ds(off, CHUNK)], idx_v)
            pltpu.async_copy(y_hbm.at[idx_v], rows_v, sem).wait()
            pltpu.sync_copy(rows_v, out_hbm.at[pl.ds(off, CHUNK)])

    return run(y, g)


def kernel(x, gate_w, cfc_w, cfc_b, cproj_w, cproj_b):
    Bx, Tx, Cx = x.shape
    x_flat = x.reshape(Bx * Tx, Cx)

    g3, w3, stats = _router(x_flat, gate_w)
    g = g3.reshape(N)
    w = w3.reshape(N)

    sum_p2 = stats[0]
    sum_rp = stats[1]
    sum_w = stats[2]
    hist = stats[3]
    s_p1mp = stats[4, 0]
    s_lse2 = stats[5, 0]

    penalty_a = s_p1mp / (N * E)
    prob_mean = sum_p2 / N
    penalty_b = 1.0 / E - jnp.mean(prob_mean * (1.0 - prob_mean))
    load = E * jnp.sum((hist / N) * (sum_rp / N))
    importance = jnp.var(sum_w) / (jnp.mean(sum_w) ** 2)
    aux = (0.001 * (penalty_a + penalty_b)
           + 0.001 * (s_lse2 / N)
           + 0.01 * load
           + 0.01 * importance).astype(jnp.float32)

    counts = jnp.minimum(hist.astype(jnp.int32), CAP)

    d, ws = _dispatch(x_flat, g, w)
    y = _ffn(counts, d, cfc_w, cfc_b, cproj_w, cproj_b, ws.reshape(RD, 1))
    out_flat = _combine(y, g)
    return out_flat.reshape(Bx, Tx, Cx), aux


# speedup vs baseline: 3.5340x; 1.0870x over previous
import functools

import jax
import jax.numpy as jnp
from jax import lax
from jax.experimental import pallas as pl
from jax.experimental.pallas import tpu as pltpu
from jax.experimental.pallas import tpu_sc as plsc

N = 8192
C = 1024
E = 8
DFF = 4096
CAP = 1536

BN = 1024
NB = N // BN

BM = CAP
BK = 1024
K = DFF // BK
RD = BM + E * CAP
EB = RD // BM

NSC, NSUB = 2, 16
NW = NSC * NSUB
TPW = N // NW
CHUNK = 64


def _router_body(x_ref, gw_ref, g_ref, w_ref, stats_ref):
    i = pl.program_id(0)

    @pl.when(i == 0)
    def _():
        stats_ref[...] = jnp.zeros((8, 8), jnp.float32)

    xb = x_ref[...]
    logits = jnp.dot(xb, gw_ref[...], preferred_element_type=jnp.float32)

    l2 = logits / 1.66
    m2 = jnp.max(l2, axis=-1, keepdims=True)
    e2 = jnp.exp(l2 - m2)
    p2 = e2 / jnp.sum(e2, axis=-1, keepdims=True)
    s_p1mp = jnp.sum(p2 * (1.0 - p2))
    sum_p2 = jnp.sum(p2, axis=0, keepdims=True)

    lm = jnp.max(logits, axis=-1, keepdims=True)
    el = jnp.exp(logits - lm)
    se = jnp.sum(el, axis=-1, keepdims=True)
    lse = lm + jnp.log(se)
    s_lse2 = jnp.sum(lse * lse)

    rp = el / se
    w = jnp.max(rp, axis=-1)
    iota_e = lax.broadcasted_iota(jnp.int32, (BN, E), 1)
    eq = rp == w[:, None]
    idx = jnp.min(jnp.where(eq, iota_e, E), axis=-1)
    onehot = (iota_e == idx[:, None]).astype(jnp.float32)

    sum_rp = jnp.sum(rp, axis=0, keepdims=True)
    sum_w = jnp.sum(onehot * w[:, None], axis=0, keepdims=True)
    blk_hist = jnp.sum(onehot, axis=0, keepdims=True)

    r_io = lax.broadcasted_iota(jnp.int32, (BN, BN), 0)
    c_io = lax.broadcasted_iota(jnp.int32, (BN, BN), 1)
    tri = (r_io >= c_io).astype(jnp.float32)
    incl = jnp.dot(tri, onehot, preferred_element_type=jnp.float32)

    offs = stats_ref[3:4, :]
    rank = incl + offs
    rank_t = jnp.sum(rank * onehot, axis=-1)
    kept = rank_t <= CAP
    slot = jnp.where(kept, BM + idx * CAP + rank_t.astype(jnp.int32) - 1, 0)

    g_ref[...] = slot.reshape(1, 1, BN)
    w_ref[...] = w.reshape(1, 1, BN)

    stats_ref[0:1, :] += sum_p2
    stats_ref[1:2, :] += sum_rp
    stats_ref[2:3, :] += sum_w
    stats_ref[3:4, :] = offs + blk_hist
    stats_ref[4:5, :] += jnp.full((1, 8), s_p1mp, jnp.float32)
    stats_ref[5:6, :] += jnp.full((1, 8), s_lse2, jnp.float32)


def _router(x_flat, gate_w):
    return pl.pallas_call(
        _router_body,
        grid=(NB,),
        in_specs=[
            pl.BlockSpec((BN, C), lambda i: (i, 0)),
            pl.BlockSpec((C, E), lambda i: (0, 0)),
        ],
        out_specs=[
            pl.BlockSpec((1, 1, BN), lambda i: (i, 0, 0)),
            pl.BlockSpec((1, 1, BN), lambda i: (i, 0, 0)),
            pl.BlockSpec((8, 8), lambda i: (0, 0)),
        ],
        out_shape=[
            jax.ShapeDtypeStruct((NB, 1, BN), jnp.int32),
            jax.ShapeDtypeStruct((NB, 1, BN), jnp.float32),
            jax.ShapeDtypeStruct((8, 8), jnp.float32),
        ],
    )(x_flat, gate_w)


def _dispatch(x_flat, g, w):
    mesh = plsc.VectorSubcoreMesh(
        core_axis_name="c", subcore_axis_name="s",
        num_cores=NSC, num_subcores=NSUB)

    @functools.partial(
        pl.kernel, mesh=mesh,
        out_type=[
            jax.ShapeDtypeStruct((RD, C), jnp.float32),
            jax.ShapeDtypeStruct((RD,), jnp.float32),
        ],
        scratch_types=[
            pltpu.VMEM((CHUNK,), jnp.int32),
            pltpu.VMEM((CHUNK, C), jnp.float32),
            pltpu.VMEM((CHUNK,), jnp.float32),
            pltpu.SemaphoreType.DMA,
        ],
    )
    def run(x_hbm, g_hbm, w_hbm, d_out, ws_out, idx_v, rows_v, wv, sem):
        wid = lax.axis_index("s") * NSC + lax.axis_index("c")
        base = wid * TPW
        for cb in range(TPW // CHUNK):
            off = base + cb * CHUNK
            pltpu.sync_copy(g_hbm.at[pl.ds(off, CHUNK)], idx_v)
            pltpu.sync_copy(x_hbm.at[pl.ds(off, CHUNK)], rows_v)
            pltpu.async_copy(rows_v, d_out.at[idx_v], sem).wait()
            pltpu.sync_copy(w_hbm.at[pl.ds(off, CHUNK)], wv)
            pltpu.async_copy(wv, ws_out.at[idx_v], sem).wait()

    return run(x_flat, g, w)


def _ffn_body(counts_ref, d_ref, w1_ref, b1_ref, w2_ref, b2_ref, ws_ref, y_ref):
    b = pl.program_id(0)
    k = pl.program_id(1)
    e = jnp.clip(b - 1, 0, E - 1)
    active = jnp.logical_and(b >= 1, counts_ref[e] > 0)

    @pl.when(active)
    def _():
        x = d_ref[...]
        h = jnp.dot(x, w1_ref[0], preferred_element_type=jnp.float32)
        h = h + b1_ref[0]
        h = jax.nn.gelu(h, approximate=True)
        contrib = jnp.dot(h, w2_ref[0], preferred_element_type=jnp.float32)

        @pl.when(k == 0)
        def _():
            y_ref[...] = contrib

        @pl.when(k != 0)
        def _():
            y_ref[...] += contrib

        @pl.when(k == K - 1)
        def _():
            y_ref[...] = (y_ref[...] + b2_ref[0]) * ws_ref[...]

    @pl.when(jnp.logical_and(jnp.logical_not(active), k == K - 1))
    def _():
        y_ref[...] = jnp.zeros((BM, C), jnp.float32)


def _ffn(counts, d, cfc_w, cfc_b, cproj_w, cproj_b, ws2d):
    grid_spec = pltpu.PrefetchScalarGridSpec(
        num_scalar_prefetch=1,
        grid=(EB, K),
        in_specs=[
            pl.BlockSpec((BM, C), lambda b, k, cnt: (b, 0)),
            pl.BlockSpec((1, C, BK), lambda b, k, cnt: (lax.max(b - 1, 0), 0, k)),
            pl.BlockSpec((1, 1, BK), lambda b, k, cnt: (lax.max(b - 1, 0), 0, k)),
            pl.BlockSpec((1, BK, C), lambda b, k, cnt: (lax.max(b - 1, 0), k, 0)),
            pl.BlockSpec((1, 1, C), lambda b, k, cnt: (lax.max(b - 1, 0), 0, 0)),
            pl.BlockSpec((BM, 1), lambda b, k, cnt: (b, 0)),
        ],
        out_specs=pl.BlockSpec((BM, C), lambda b, k, cnt: (b, 0)),
    )
    return pl.pallas_call(
        _ffn_body,
        grid_spec=grid_spec,
        out_shape=jax.ShapeDtypeStruct((RD, C), jnp.float32),
        compiler_params=pltpu.CompilerParams(
            dimension_semantics=("arbitrary", "arbitrary")),
    )(counts, d, cfc_w, cfc_b.reshape(E, 1, DFF), cproj_w,
      cproj_b.reshape(E, 1, C), ws2d)


def _combine(y, g):
    mesh = plsc.VectorSubcoreMesh(
        core_axis_name="c", subcore_axis_name="s",
        num_cores=NSC, num_subcores=NSUB)

    @functools.partial(
        pl.kernel, mesh=mesh,
        out_type=jax.ShapeDtypeStruct((N, C), jnp.float32),
        scratch_types=[
            pltpu.VMEM((CHUNK,), jnp.int32),
            pltpu.VMEM((CHUNK, C), jnp.float32),
            pltpu.SemaphoreType.DMA,
        ],
    )
    def run(y_hbm, g_hbm, out_hbm, idx_v, rows_v, sem):
        wid = lax.axis_index("s") * NSC + lax.axis_index("c")
        base = wid * TPW
        for cb in range(TPW // CHUNK):
            off = base + cb * CHUNK
            pltpu.sync_copy(g_hbm.at[pl.ds(off, CHUNK)], idx_v)
            pltpu.async_copy(y_hbm.at[idx_v], rows_v, sem).wait()
            pltpu.sync_copy(rows_v, out_hbm.at[pl.ds(off, CHUNK)])

    return run(y, g)


def kernel(x, gate_w, cfc_w, cfc_b, cproj_w, cproj_b):
    Bx, Tx, Cx = x.shape
    x_flat = x.reshape(Bx * Tx, Cx)

    g3, w3, stats = _router(x_flat, gate_w)
    g = g3.reshape(N)
    w = w3.reshape(N)

    sum_p2 = stats[0]
    sum_rp = stats[1]
    sum_w = stats[2]
    hist = stats[3]
    s_p1mp = stats[4, 0]
    s_lse2 = stats[5, 0]

    penalty_a = s_p1mp / (N * E)
    prob_mean = sum_p2 / N
    penalty_b = 1.0 / E - jnp.mean(prob_mean * (1.0 - prob_mean))
    load = E * jnp.sum((hist / N) * (sum_rp / N))
    importance = jnp.var(sum_w) / (jnp.mean(sum_w) ** 2)
    aux = (0.001 * (penalty_a + penalty_b)
           + 0.001 * (s_lse2 / N)
           + 0.01 * load
           + 0.01 * importance).astype(jnp.float32)

    counts = jnp.minimum(hist.astype(jnp.int32), CAP)

    d, ws = _dispatch(x_flat, g, w)
    y = _ffn(counts, d, cfc_w, cfc_b, cproj_w, cproj_b, ws.reshape(RD, 1))
    out_flat = _combine(y, g)
    return out_flat.reshape(Bx, Tx, Cx), aux
